# baseline (speedup 1.0000x reference)
import jax
import jax.numpy as jnp
from jax.experimental import pallas as pl

N = 16384
K = 1000
KP = 1024
F = 16
TN = 2048
G = N // TN


def _body(x_ref, c_ref, c2_ref, dist_ref, asg_ref, ine_ref):
    x = x_ref[...]
    c = c_ref[...]
    c2 = c2_ref[...]
    x2 = jnp.sum(x * x, axis=1, keepdims=True)
    xc = jax.lax.dot_general(
        x, c, (((1,), (1,)), ((), ())), preferred_element_type=jnp.float32
    )
    d2 = jnp.maximum(x2 + c2 - 2.0 * xc, 0.0)
    m = jnp.min(d2, axis=1)
    idx = jnp.argmin(d2, axis=1).astype(jnp.int32)
    dist_ref[...] = jnp.pad(jnp.sqrt(d2), ((0, 0), (0, KP - K)))
    asg_ref[0, 0, :] = idx
    ine_ref[0, 0, :] = m


def kernel(data, centroids):
    c2 = jnp.sum(centroids * centroids, axis=1)[None, :]

    distances, asg3, ine3 = pl.pallas_call(
        _body,
        grid=(G,),
        in_specs=[
            pl.BlockSpec((TN, F), lambda i: (i, 0)),
            pl.BlockSpec((K, F), lambda i: (0, 0)),
            pl.BlockSpec((1, K), lambda i: (0, 0)),
        ],
        out_specs=[
            pl.BlockSpec((TN, KP), lambda i: (i, 0)),
            pl.BlockSpec((1, 1, TN), lambda i: (i, 0, 0)),
            pl.BlockSpec((1, 1, TN), lambda i: (i, 0, 0)),
        ],
        out_shape=[
            jax.ShapeDtypeStruct((N, KP), jnp.float32),
            jax.ShapeDtypeStruct((G, 1, TN), jnp.int32),
            jax.ShapeDtypeStruct((G, 1, TN), jnp.float32),
        ],
    )(data, centroids, c2)
    return distances[:, :K], asg3.reshape(N), ine3.reshape(N)


# speedup vs baseline: 1.0267x; 1.0267x over previous
import jax
import jax.numpy as jnp
from jax.experimental import pallas as pl
from jax.experimental.pallas import tpu as pltpu

N = 16384
K = 1000
F = 16
TN = 2048
G = N // TN
NS = 4
SR = TN // NS


def _body(x_ref, c_ref, c2_ref, dist_hbm, asg_ref, ine_ref, scr_ref, sem_ref):
    i = pl.program_id(0)
    buf = jax.lax.rem(i, 2)

    def _stripe_copy(step, b, s):
        src = scr_ref.at[b, pl.ds(s * SR, SR), :]
        dst = dist_hbm.at[pl.ds(step * TN + s * SR, SR), :]
        return pltpu.make_async_copy(src, dst, sem_ref.at[b, s])

    @pl.when(i >= 2)
    def _():
        for s in range(NS):
            _stripe_copy(i - 2, buf, s).wait()

    x = x_ref[...]
    c = c_ref[...]
    c2 = c2_ref[...]
    x2 = jnp.sum(x * x, axis=1, keepdims=True)
    xc = jax.lax.dot_general(
        x, c, (((1,), (1,)), ((), ())), preferred_element_type=jnp.float32
    )
    d2 = jnp.maximum(x2 + c2 - 2.0 * xc, 0.0)
    m = jnp.min(d2, axis=1)
    idx = jnp.argmin(d2, axis=1).astype(jnp.int32)
    scr_ref[buf] = jnp.sqrt(d2)
    asg_ref[0, 0, :] = idx
    ine_ref[0, 0, :] = m

    for s in range(NS):
        _stripe_copy(i, buf, s).start()

    @pl.when(i == G - 1)
    def _():
        for s in range(NS):
            _stripe_copy(i - 1, 1 - buf, s).wait()
            _stripe_copy(i, buf, s).wait()


def kernel(data, centroids):
    c2 = jnp.sum(centroids * centroids, axis=1)[None, :]

    distances, asg3, ine3 = pl.pallas_call(
        _body,
        grid=(G,),
        in_specs=[
            pl.BlockSpec((TN, F), lambda i: (i, 0)),
            pl.BlockSpec((K, F), lambda i: (0, 0)),
            pl.BlockSpec((1, K), lambda i: (0, 0)),
        ],
        out_specs=[
            pl.BlockSpec(memory_space=pl.ANY),
            pl.BlockSpec((1, 1, TN), lambda i: (i, 0, 0)),
            pl.BlockSpec((1, 1, TN), lambda i: (i, 0, 0)),
        ],
        out_shape=[
            jax.ShapeDtypeStruct((N, K), jnp.float32),
            jax.ShapeDtypeStruct((G, 1, TN), jnp.int32),
            jax.ShapeDtypeStruct((G, 1, TN), jnp.float32),
        ],
        scratch_shapes=[
            pltpu.VMEM((2, TN, K), jnp.float32),
            pltpu.SemaphoreType.DMA((2, NS)),
        ],
    )(data, centroids, c2)
    return distances, asg3.reshape(N), ine3.reshape(N)


# speedup vs baseline: 1.0302x; 1.0034x over previous
import jax
import jax.numpy as jnp
from jax.experimental import pallas as pl
from jax.experimental.pallas import tpu as pltpu

N = 16384
K = 1000
F = 16
TN = 2048
G = N // TN
KA = 896


def _body(x_ref, c_ref, c2_ref, dist_hbm, asg_ref, ine_ref, scr_ref, sem_ref):
    i = pl.program_id(0)
    buf = jax.lax.rem(i, 2)

    def _copies(step, b):
        rows = pl.ds(step * TN, TN)
        main = pltpu.make_async_copy(
            scr_ref.at[b, :, pl.ds(0, KA)],
            dist_hbm.at[rows, pl.ds(0, KA)],
            sem_ref.at[b, 0],
        )
        tail = pltpu.make_async_copy(
            scr_ref.at[b, :, pl.ds(KA, K - KA)],
            dist_hbm.at[rows, pl.ds(KA, K - KA)],
            sem_ref.at[b, 1],
        )
        return main, tail

    @pl.when(i >= 2)
    def _():
        for cp in _copies(i - 2, buf):
            cp.wait()

    x = x_ref[...]
    c = c_ref[...]
    c2 = c2_ref[...]
    x2 = jnp.sum(x * x, axis=1, keepdims=True)
    xc = jax.lax.dot_general(
        x, c, (((1,), (1,)), ((), ())), preferred_element_type=jnp.float32
    )
    d2 = jnp.maximum(x2 + c2 - 2.0 * xc, 0.0)
    m = jnp.min(d2, axis=1)
    idx = jnp.argmin(d2, axis=1).astype(jnp.int32)
    scr_ref[buf] = jnp.sqrt(d2)
    asg_ref[0, 0, :] = idx
    ine_ref[0, 0, :] = m

    for cp in _copies(i, buf):
        cp.start()

    @pl.when(i == G - 1)
    def _():
        for cp in _copies(i - 1, 1 - buf):
            cp.wait()
        for cp in _copies(i, buf):
            cp.wait()


def kernel(data, centroids):
    c2 = jnp.sum(centroids * centroids, axis=1)[None, :]

    distances, asg3, ine3 = pl.pallas_call(
        _body,
        grid=(G,),
        in_specs=[
            pl.BlockSpec((TN, F), lambda i: (i, 0)),
            pl.BlockSpec((K, F), lambda i: (0, 0)),
            pl.BlockSpec((1, K), lambda i: (0, 0)),
        ],
        out_specs=[
            pl.BlockSpec(memory_space=pl.ANY),
            pl.BlockSpec((1, 1, TN), lambda i: (i, 0, 0)),
            pl.BlockSpec((1, 1, TN), lambda i: (i, 0, 0)),
        ],
        out_shape=[
            jax.ShapeDtypeStruct((N, K), jnp.float32),
            jax.ShapeDtypeStruct((G, 1, TN), jnp.int32),
            jax.ShapeDtypeStruct((G, 1, TN), jnp.float32),
        ],
        scratch_shapes=[
            pltpu.VMEM((2, TN, K), jnp.float32),
            pltpu.SemaphoreType.DMA((2, 2)),
        ],
    )(data, centroids, c2)
    return distances, asg3.reshape(N), ine3.reshape(N)
